# Initial kernel scaffold; baseline (speedup 1.0000x reference)
import jax
import jax.numpy as jnp
from jax.experimental import pallas as pl


def kernel(x, Wg, W1, b1, W2, b2):
    raise NotImplementedError("write your pallas kernel here")


# speedup vs baseline: 1.3288x; 1.3288x over previous
import functools

import jax
import jax.numpy as jnp
from jax import lax
from jax.experimental import pallas as pl
from jax.experimental.pallas import tpu as pltpu
from jax.experimental.pallas import tpu_sc as plsc

N = 2048
D = 1024
F = 2048
E = 8
TM = 256
NT = N * 2 // TM + (E - 1)
A_PAD = NT * TM
NC, NS = 2, 16
NW = NC * NS
DISP_ROWS = A_PAD // NW
COMB_TOK = N // NW

def _sc_mesh():
    return plsc.VectorSubcoreMesh(core_axis_name="c", subcore_axis_name="s")


def _router_body(x_ref, wg_ref, ind_ref, prob_ref):
    logits = jnp.dot(x_ref[...], wg_ref[...], preferred_element_type=jnp.float32)
    col = lax.broadcasted_iota(jnp.int32, logits.shape, 1)
    v0 = jnp.max(logits, axis=1, keepdims=True)
    i0 = jnp.min(jnp.where(logits == v0, col, E), axis=1, keepdims=True)
    masked = jnp.where(col == i0, -jnp.inf, logits)
    v1 = jnp.max(masked, axis=1, keepdims=True)
    i1 = jnp.min(jnp.where(masked == v1, col, E), axis=1, keepdims=True)
    p0 = 1.0 / (1.0 + jnp.exp(v1 - v0))
    ind_ref[...] = jnp.concatenate([i0, i1], axis=1)
    prob_ref[...] = jnp.concatenate([p0, 1.0 - p0], axis=1)


def _run_router(xf, Wg):
    return pl.pallas_call(
        _router_body,
        grid=(N // TM,),
        in_specs=[
            pl.BlockSpec((TM, D), lambda t: (t, 0)),
            pl.BlockSpec((D, E), lambda t: (0, 0)),
        ],
        out_specs=[
            pl.BlockSpec((TM, 2), lambda t: (t, 0)),
            pl.BlockSpec((TM, 2), lambda t: (t, 0)),
        ],
        out_shape=[
            jax.ShapeDtypeStruct((N, 2), jnp.int32),
            jax.ShapeDtypeStruct((N, 2), jnp.float32),
        ],
    )(xf, Wg)


def _dispatch_body(tok_hbm, xf_hbm, xs_hbm, idx_v, rows_v, sem):
    wid = lax.axis_index("s") * NC + lax.axis_index("c")
    base = wid * DISP_ROWS
    pltpu.sync_copy(tok_hbm.at[pl.ds(base, DISP_ROWS)], idx_v)
    for off, sz in ((0, 96), (96, 88)):
        pltpu.async_copy(
            xf_hbm.at[idx_v.at[pl.ds(off, sz)]], rows_v.at[pl.ds(0, sz)], sem
        ).wait()
        pltpu.sync_copy(rows_v.at[pl.ds(0, sz)], xs_hbm.at[pl.ds(base + off, sz)])


def _run_dispatch(tok_sorted, xf):
    return pl.kernel(
        _dispatch_body,
        out_type=jax.ShapeDtypeStruct((A_PAD, D), jnp.float32),
        mesh=_sc_mesh(),
        scratch_types=[
            pltpu.VMEM((DISP_ROWS,), jnp.int32),
            pltpu.VMEM((96, D), jnp.float32),
            pltpu.SemaphoreType.DMA,
        ],
    )(tok_sorted, xf)


def _ffn_body(eot_ref, valid_ref, xs_ref, ps_ref, w1_ref, b1_ref, w2_ref,
              b2_ref, out_ref):
    t = pl.program_id(0)

    @pl.when(valid_ref[t] == 1)
    def _():
        x = xs_ref[...].astype(jnp.bfloat16)
        h = jnp.dot(x, w1_ref[0], preferred_element_type=jnp.float32)
        h = h + b1_ref[0]
        h = 0.5 * h * (1.0 + lax.erf(h * 0.7071067811865476))
        o = jnp.dot(h.astype(jnp.bfloat16), w2_ref[0],
                    preferred_element_type=jnp.float32)
        out_ref[...] = (o + b2_ref[0]) * ps_ref[...]

    @pl.when(valid_ref[t] == 0)
    def _():
        out_ref[...] = jnp.zeros_like(out_ref)


def _run_ffn(eot, valid, xs, ps, W1b, b1, W2b, b2):
    grid_spec = pltpu.PrefetchScalarGridSpec(
        num_scalar_prefetch=2,
        grid=(NT,),
        in_specs=[
            pl.BlockSpec((TM, D), lambda t, eot, valid: (t, 0)),
            pl.BlockSpec((TM, 1), lambda t, eot, valid: (t, 0)),
            pl.BlockSpec((1, D, F), lambda t, eot, valid: (eot[t], 0, 0)),
            pl.BlockSpec((1, 1, F), lambda t, eot, valid: (eot[t], 0, 0)),
            pl.BlockSpec((1, F, D), lambda t, eot, valid: (eot[t], 0, 0)),
            pl.BlockSpec((1, 1, D), lambda t, eot, valid: (eot[t], 0, 0)),
        ],
        out_specs=pl.BlockSpec((TM, D), lambda t, eot, valid: (t, 0)),
    )
    return pl.pallas_call(
        _ffn_body,
        grid_spec=grid_spec,
        out_shape=jax.ShapeDtypeStruct((A_PAD, D), jnp.float32),
    )(eot, valid, xs, ps, W1b, b1, W2b, b2)


def _combine_body(outs_hbm, pos0_hbm, pos1_hbm, y_hbm, i0_v, i1_v, a_v, b_v,
                  y_v, sem_a, sem_b):
    wid = lax.axis_index("s") * NC + lax.axis_index("c")
    base = wid * COMB_TOK
    pltpu.sync_copy(pos0_hbm.at[pl.ds(base, COMB_TOK)], i0_v)
    pltpu.sync_copy(pos1_hbm.at[pl.ds(base, COMB_TOK)], i1_v)
    def _chunk(c, _):
        cpa = pltpu.async_copy(outs_hbm.at[i0_v.at[pl.ds(c * 16, 16)]], a_v, sem_a)
        cpb = pltpu.async_copy(outs_hbm.at[i1_v.at[pl.ds(c * 16, 16)]], b_v, sem_b)
        cpa.wait()
        cpb.wait()

        def _row(j, _):
            def _col(u, _):
                sl = pl.ds(u * 16, 16)
                y_v[j, sl] = a_v[j, sl] + b_v[j, sl]
                return 0
            return lax.fori_loop(0, D // 16, _col, 0, unroll=8)

        lax.fori_loop(0, 16, _row, 0)
        pltpu.sync_copy(y_v, y_hbm.at[pl.ds(base + c * 16, 16)])
        return 0

    lax.fori_loop(0, COMB_TOK // 16, _chunk, 0)


def _run_combine(outs, pos0, pos1):
    return pl.kernel(
        _combine_body,
        out_type=jax.ShapeDtypeStruct((N, D), jnp.float32),
        mesh=_sc_mesh(),
        scratch_types=[
            pltpu.VMEM((COMB_TOK,), jnp.int32),
            pltpu.VMEM((COMB_TOK,), jnp.int32),
            pltpu.VMEM((16, D), jnp.float32),
            pltpu.VMEM((16, D), jnp.float32),
            pltpu.VMEM((16, D), jnp.float32),
            pltpu.SemaphoreType.DMA,
            pltpu.SemaphoreType.DMA,
        ],
    )(outs, pos0, pos1)


def kernel(x, Wg, W1, b1, W2, b2):
    Bq, Tq, C = x.shape
    xf = x.reshape(N, D)

    indices, probs = _run_router(xf, Wg)

    toks = jnp.arange(N, dtype=jnp.int32)
    onehot = ((indices[:, 0:1] == jnp.arange(E)[None, :]).astype(jnp.int32)
              + (indices[:, 1:2] == jnp.arange(E)[None, :]).astype(jnp.int32))
    incl = jnp.cumsum(onehot, axis=0)
    excl = incl - onehot
    cnt = incl[-1]
    tiles_e = (cnt + TM - 1) // TM
    tile_start = jnp.cumsum(tiles_e) - tiles_e
    row_start = TM * tile_start
    total_tiles = jnp.sum(tiles_e)
    pos = row_start[indices] + jnp.take_along_axis(excl, indices, axis=1)
    tok_sorted = (jnp.zeros((A_PAD,), jnp.int32)
                  .at[pos[:, 0]].set(toks).at[pos[:, 1]].set(toks))
    ps_sorted = (jnp.zeros((A_PAD,), jnp.float32)
                 .at[pos[:, 0]].set(probs[:, 0])
                 .at[pos[:, 1]].set(probs[:, 1])).reshape(A_PAD, 1)
    trange = jnp.arange(NT, dtype=jnp.int32)
    eot = jnp.clip(jnp.sum((trange[:, None] >= tile_start[None, :]).astype(jnp.int32),
                           axis=1) - 1, 0, E - 1).astype(jnp.int32)
    valid = (trange < total_tiles).astype(jnp.int32)

    xs = _run_dispatch(tok_sorted, xf)
    outs = _run_ffn(eot, valid, xs, ps_sorted,
                    W1.astype(jnp.bfloat16), b1.reshape(E, 1, F),
                    W2.astype(jnp.bfloat16), b2.reshape(E, 1, D))
    y = _run_combine(outs, pos[:, 0].astype(jnp.int32), pos[:, 1].astype(jnp.int32))
    return (y.reshape(Bq, Tq, C), indices)


# speedup vs baseline: 1.3511x; 1.0167x over previous
import functools

import jax
import jax.numpy as jnp
from jax import lax
from jax.experimental import pallas as pl
from jax.experimental.pallas import tpu as pltpu
from jax.experimental.pallas import tpu_sc as plsc

N = 2048
D = 1024
F = 2048
E = 8
TM = 256
NT = N * 2 // TM + (E - 1)
A_PAD = NT * TM
NC, NS = 2, 16
NW = NC * NS
DISP_ROWS = A_PAD // NW
COMB_TOK = N // NW

def _sc_mesh():
    return plsc.VectorSubcoreMesh(core_axis_name="c", subcore_axis_name="s")


def _router_body(x_ref, wg_ref, ind_ref, prob_ref):
    logits = jnp.dot(x_ref[...], wg_ref[...], preferred_element_type=jnp.float32)
    col = lax.broadcasted_iota(jnp.int32, logits.shape, 1)
    v0 = jnp.max(logits, axis=1, keepdims=True)
    i0 = jnp.min(jnp.where(logits == v0, col, E), axis=1, keepdims=True)
    masked = jnp.where(col == i0, -jnp.inf, logits)
    v1 = jnp.max(masked, axis=1, keepdims=True)
    i1 = jnp.min(jnp.where(masked == v1, col, E), axis=1, keepdims=True)
    p0 = 1.0 / (1.0 + jnp.exp(v1 - v0))
    ind_ref[...] = jnp.concatenate([i0, i1], axis=1)
    prob_ref[...] = jnp.concatenate([p0, 1.0 - p0], axis=1)


def _run_router(xf, Wg):
    return pl.pallas_call(
        _router_body,
        grid=(N // TM,),
        in_specs=[
            pl.BlockSpec((TM, D), lambda t: (t, 0)),
            pl.BlockSpec((D, E), lambda t: (0, 0)),
        ],
        out_specs=[
            pl.BlockSpec((TM, 2), lambda t: (t, 0)),
            pl.BlockSpec((TM, 2), lambda t: (t, 0)),
        ],
        out_shape=[
            jax.ShapeDtypeStruct((N, 2), jnp.int32),
            jax.ShapeDtypeStruct((N, 2), jnp.float32),
        ],
    )(xf, Wg)


_DCH = ((0, 48), (48, 48), (96, 48), (144, 40))


def _dispatch_body(tok_hbm, xf_hbm, xs_hbm, idx_v, buf0, buf1, gs0, gs1,
                   ss0, ss1):
    wid = lax.axis_index("s") * NC + lax.axis_index("c")
    base = wid * DISP_ROWS
    pltpu.sync_copy(tok_hbm.at[pl.ds(base, DISP_ROWS)], idx_v)
    bufs, gsems, ssems = (buf0, buf1), (gs0, gs1), (ss0, ss1)

    def gather(c):
        off, sz = _DCH[c]
        return pltpu.async_copy(
            xf_hbm.at[idx_v.at[pl.ds(off, sz)]],
            bufs[c % 2].at[pl.ds(0, sz)], gsems[c % 2])

    def store(c):
        off, sz = _DCH[c]
        return pltpu.async_copy(
            bufs[c % 2].at[pl.ds(0, sz)],
            xs_hbm.at[pl.ds(base + off, sz)], ssems[c % 2])

    g = gather(0)
    stores = []
    for c in range(4):
        g.wait()
        stores.append(store(c))
        if c + 1 < 4:
            if c >= 1:
                stores[c - 1].wait()
            g = gather(c + 1)
    stores[2].wait()
    stores[3].wait()


def _run_dispatch(tok_sorted, xf):
    return pl.kernel(
        _dispatch_body,
        out_type=jax.ShapeDtypeStruct((A_PAD, D), jnp.float32),
        mesh=_sc_mesh(),
        scratch_types=[
            pltpu.VMEM((DISP_ROWS,), jnp.int32),
            pltpu.VMEM((48, D), jnp.float32),
            pltpu.VMEM((48, D), jnp.float32),
            pltpu.SemaphoreType.DMA,
            pltpu.SemaphoreType.DMA,
            pltpu.SemaphoreType.DMA,
            pltpu.SemaphoreType.DMA,
        ],
    )(tok_sorted, xf)


def _ffn_body(eot_ref, valid_ref, xs_ref, ps_ref, w1_ref, b1_ref, w2_ref,
              b2_ref, out_ref):
    t = pl.program_id(0)

    @pl.when(valid_ref[t] == 1)
    def _():
        x = xs_ref[...].astype(jnp.bfloat16)
        h = jnp.dot(x, w1_ref[0], preferred_element_type=jnp.float32)
        h = h + b1_ref[0]
        h = 0.5 * h * (1.0 + lax.erf(h * 0.7071067811865476))
        o = jnp.dot(h.astype(jnp.bfloat16), w2_ref[0],
                    preferred_element_type=jnp.float32)
        out_ref[...] = (o + b2_ref[0]) * ps_ref[...]

    @pl.when(valid_ref[t] == 0)
    def _():
        out_ref[...] = jnp.zeros_like(out_ref)


def _run_ffn(eot, valid, xs, ps, W1b, b1, W2b, b2):
    grid_spec = pltpu.PrefetchScalarGridSpec(
        num_scalar_prefetch=2,
        grid=(NT,),
        in_specs=[
            pl.BlockSpec((TM, D), lambda t, eot, valid: (t, 0)),
            pl.BlockSpec((TM, 1), lambda t, eot, valid: (t, 0)),
            pl.BlockSpec((1, D, F), lambda t, eot, valid: (eot[t], 0, 0)),
            pl.BlockSpec((1, 1, F), lambda t, eot, valid: (eot[t], 0, 0)),
            pl.BlockSpec((1, F, D), lambda t, eot, valid: (eot[t], 0, 0)),
            pl.BlockSpec((1, 1, D), lambda t, eot, valid: (eot[t], 0, 0)),
        ],
        out_specs=pl.BlockSpec((TM, D), lambda t, eot, valid: (t, 0)),
    )
    return pl.pallas_call(
        _ffn_body,
        grid_spec=grid_spec,
        out_shape=jax.ShapeDtypeStruct((A_PAD, D), jnp.float32),
    )(eot, valid, xs, ps, W1b, b1, W2b, b2)


def _combine_body(outs_hbm, pos0_hbm, pos1_hbm, y_hbm, i0_v, i1_v,
                  a0, b0, a1, b1, y0, y1,
                  sa0, sb0, sa1, sb1, sy0, sy1):
    wid = lax.axis_index("s") * NC + lax.axis_index("c")
    base = wid * COMB_TOK
    pltpu.sync_copy(pos0_hbm.at[pl.ds(base, COMB_TOK)], i0_v)
    pltpu.sync_copy(pos1_hbm.at[pl.ds(base, COMB_TOK)], i1_v)
    av, bv, yv = (a0, a1), (b0, b1), (y0, y1)
    sas, sbs, sys_ = (sa0, sa1), (sb0, sb1), (sy0, sy1)
    nch = COMB_TOK // 16

    def gathers(c):
        p = c % 2
        return (pltpu.async_copy(outs_hbm.at[i0_v.at[pl.ds(c * 16, 16)]],
                                 av[p], sas[p]),
                pltpu.async_copy(outs_hbm.at[i1_v.at[pl.ds(c * 16, 16)]],
                                 bv[p], sbs[p]))

    g = gathers(0)
    stores = []
    for c in range(nch):
        p = c % 2
        g[0].wait()
        g[1].wait()
        if c >= 2:
            stores[c - 2].wait()
        if c + 1 < nch:
            g = gathers(c + 1)
        a_v, b_v, y_v = av[p], bv[p], yv[p]

        def _row(j, _, a_v=a_v, b_v=b_v, y_v=y_v):
            def _blk(i, _):
                b_off = i * 256
                for t in range(16):
                    sl = pl.ds(b_off + t * 16, 16)
                    y_v[j, sl] = a_v[j, sl] + b_v[j, sl]
                return 0
            return lax.fori_loop(0, D // 256, _blk, 0)

        lax.fori_loop(0, 16, _row, 0)
        stores.append(pltpu.async_copy(
            y_v, y_hbm.at[pl.ds(base + c * 16, 16)], sys_[p]))
    stores[nch - 2].wait()
    stores[nch - 1].wait()


def _run_combine(outs, pos0, pos1):
    return pl.kernel(
        _combine_body,
        out_type=jax.ShapeDtypeStruct((N, D), jnp.float32),
        mesh=_sc_mesh(),
        scratch_types=[
            pltpu.VMEM((COMB_TOK,), jnp.int32),
            pltpu.VMEM((COMB_TOK,), jnp.int32),
            pltpu.VMEM((16, D), jnp.float32),
            pltpu.VMEM((16, D), jnp.float32),
            pltpu.VMEM((16, D), jnp.float32),
            pltpu.VMEM((16, D), jnp.float32),
            pltpu.VMEM((16, D), jnp.float32),
            pltpu.VMEM((16, D), jnp.float32),
            pltpu.SemaphoreType.DMA,
            pltpu.SemaphoreType.DMA,
            pltpu.SemaphoreType.DMA,
            pltpu.SemaphoreType.DMA,
            pltpu.SemaphoreType.DMA,
            pltpu.SemaphoreType.DMA,
        ],
    )(outs, pos0, pos1)


def kernel(x, Wg, W1, b1, W2, b2):
    Bq, Tq, C = x.shape
    xf = x.reshape(N, D)

    indices, probs = _run_router(xf, Wg)

    toks = jnp.arange(N, dtype=jnp.int32)
    onehot = ((indices[:, 0:1] == jnp.arange(E)[None, :]).astype(jnp.int32)
              + (indices[:, 1:2] == jnp.arange(E)[None, :]).astype(jnp.int32))
    incl = jnp.cumsum(onehot, axis=0)
    excl = incl - onehot
    cnt = incl[-1]
    tiles_e = (cnt + TM - 1) // TM
    tile_start = jnp.cumsum(tiles_e) - tiles_e
    row_start = TM * tile_start
    total_tiles = jnp.sum(tiles_e)
    pos = row_start[indices] + jnp.take_along_axis(excl, indices, axis=1)
    tok_sorted = (jnp.zeros((A_PAD,), jnp.int32)
                  .at[pos[:, 0]].set(toks).at[pos[:, 1]].set(toks))
    ps_sorted = (jnp.zeros((A_PAD,), jnp.float32)
                 .at[pos[:, 0]].set(probs[:, 0])
                 .at[pos[:, 1]].set(probs[:, 1])).reshape(A_PAD, 1)
    trange = jnp.arange(NT, dtype=jnp.int32)
    eot = jnp.clip(jnp.sum((trange[:, None] >= tile_start[None, :]).astype(jnp.int32),
                           axis=1) - 1, 0, E - 1).astype(jnp.int32)
    valid = (trange < total_tiles).astype(jnp.int32)

    xs = _run_dispatch(tok_sorted, xf)
    outs = _run_ffn(eot, valid, xs, ps_sorted,
                    W1.astype(jnp.bfloat16), b1.reshape(E, 1, F),
                    W2.astype(jnp.bfloat16), b2.reshape(E, 1, D))
    y = _run_combine(outs, pos[:, 0].astype(jnp.int32), pos[:, 1].astype(jnp.int32))
    return (y.reshape(Bq, Tq, C), indices)
